# baseline (device time: 100534 ns/iter reference)
import jax
import jax.numpy as jnp
from jax import lax
from jax.experimental import pallas as pl
from jax.experimental.pallas import tpu as pltpu

N_DEV = 4
B = 2
SQ = 512
SKV = 512
HQ_PER = 8
DH = 64
D_MODEL = 768
CHUNK = HQ_PER * DH
BLK = 64
NEG = -1e9


def kernel(x, Wq, K_ext, V_ext, Wo):
    K2 = K_ext.reshape(B, SKV, CHUNK)
    V2 = V_ext.reshape(B, SKV, CHUNK)

    def body(x_ref, wq_ref, k_ref, v_ref, wo_ref, out_ref,
             comm_ref, send_sems, recv_sems):
        my = lax.axis_index("i")
        left = lax.rem(my + N_DEV - 1, N_DEV)
        right = lax.rem(my + 1, N_DEV)

        barrier_sem = pltpu.get_barrier_semaphore()
        for nbr in (left, right):
            pl.semaphore_signal(
                barrier_sem, inc=1,
                device_id=(nbr,), device_id_type=pl.DeviceIdType.MESH,
            )
        pl.semaphore_wait(barrier_sem, 2)

        row_blk = lax.broadcasted_iota(jnp.int32, (SQ, SKV), 0) // BLK
        col_blk = lax.broadcasted_iota(jnp.int32, (SQ, SKV), 1) // BLK
        mask = col_blk <= row_blk

        for b in range(B):
            q_all = lax.dot_general(
                x_ref[b], wq_ref[:, pl.ds(my * CHUNK, CHUNK)],
                (((1,), (0,)), ((), ())),
                preferred_element_type=jnp.float32,
            )
            for h in range(HQ_PER):
                q_h = q_all[:, h * DH:(h + 1) * DH]
                k_h = k_ref[b, :, h * DH:(h + 1) * DH]
                scores = lax.dot_general(
                    q_h, k_h, (((1,), (1,)), ((), ())),
                    preferred_element_type=jnp.float32,
                ) * 0.125
                scores = jnp.where(mask, scores, NEG)
                m = jnp.max(scores, axis=1, keepdims=True)
                w = jnp.exp(scores - m)
                w = w / jnp.sum(w, axis=1, keepdims=True)
                ctx_h = lax.dot_general(
                    w, v_ref[b, :, h * DH:(h + 1) * DH],
                    (((1,), (0,)), ((), ())),
                    preferred_element_type=jnp.float32,
                )
                comm_ref[0, b * SQ:(b + 1) * SQ, h * DH:(h + 1) * DH] = ctx_h

        for b in range(B):
            out_ref[b] = lax.dot_general(
                comm_ref[0, b * SQ:(b + 1) * SQ, :],
                wo_ref[pl.ds(my * CHUNK, CHUNK), :],
                (((1,), (0,)), ((), ())),
                preferred_element_type=jnp.float32,
            )

        for h in range(N_DEV - 1):
            rdma = pltpu.make_async_remote_copy(
                src_ref=comm_ref.at[h],
                dst_ref=comm_ref.at[h + 1],
                send_sem=send_sems.at[h],
                recv_sem=recv_sems.at[h],
                device_id=(right,),
                device_id_type=pl.DeviceIdType.MESH,
            )
            rdma.start()
            rdma.wait()
            origin = lax.rem(my + N_DEV - h - 1, N_DEV)
            wo_slice = wo_ref[pl.ds(origin * CHUNK, CHUNK), :]
            for b in range(B):
                out_ref[b] = out_ref[b] + lax.dot_general(
                    comm_ref[h + 1, b * SQ:(b + 1) * SQ, :], wo_slice,
                    (((1,), (0,)), ((), ())),
                    preferred_element_type=jnp.float32,
                )

    return pl.pallas_call(
        body,
        out_shape=jax.ShapeDtypeStruct((B, SQ, D_MODEL), jnp.float32),
        in_specs=[pl.BlockSpec(memory_space=pltpu.VMEM)] * 5,
        out_specs=pl.BlockSpec(memory_space=pltpu.VMEM),
        scratch_shapes=[
            pltpu.VMEM((N_DEV, B * SQ, CHUNK), jnp.float32),
            pltpu.SemaphoreType.DMA((N_DEV - 1,)),
            pltpu.SemaphoreType.DMA((N_DEV - 1,)),
        ],
        compiler_params=pltpu.CompilerParams(collective_id=0),
    )(x, Wq, K2, V2, Wo)


# device time: 50147 ns/iter; 2.0048x vs baseline; 2.0048x over previous
import jax
import jax.numpy as jnp
from jax import lax
from jax.experimental import pallas as pl
from jax.experimental.pallas import tpu as pltpu

N_DEV = 4
B = 2
SQ = 512
SKV = 512
HQ_PER = 8
DH = 64
D_MODEL = 768
CHUNK = HQ_PER * DH
BLK = 64
NEG = -1e9


def kernel(x, Wq, K_ext, V_ext, Wo):
    K2 = K_ext.reshape(B, SKV, CHUNK)
    V2 = V_ext.reshape(B, SKV, CHUNK)
    Wo16 = Wo.astype(jnp.bfloat16)

    def body(x_ref, wq_ref, k_ref, v_ref, wo_ref, out_ref,
             comm_ref, send_sems, recv_sems):
        my = lax.axis_index("i")

        barrier_sem = pltpu.get_barrier_semaphore()
        for r in (1, 2, 3):
            pl.semaphore_signal(
                barrier_sem, inc=1,
                device_id=(lax.rem(my + r, N_DEV),),
                device_id_type=pl.DeviceIdType.MESH,
            )
        pl.semaphore_wait(barrier_sem, N_DEV - 1)

        row_blk = lax.broadcasted_iota(jnp.int32, (SQ, SKV), 0) // BLK
        col_blk = lax.broadcasted_iota(jnp.int32, (SQ, SKV), 1) // BLK
        mask = col_blk <= row_blk

        for b in range(B):
            q_all = lax.dot_general(
                x_ref[b], wq_ref[:, pl.ds(my * CHUNK, CHUNK)],
                (((1,), (0,)), ((), ())),
                preferred_element_type=jnp.float32,
            )
            for h in range(HQ_PER):
                q_h = q_all[:, h * DH:(h + 1) * DH]
                k_h = k_ref[b, :, h * DH:(h + 1) * DH]
                scores = lax.dot_general(
                    q_h, k_h, (((1,), (1,)), ((), ())),
                    preferred_element_type=jnp.float32,
                ) * 0.125
                scores = jnp.where(mask, scores, NEG)
                m = jnp.max(scores, axis=1, keepdims=True)
                w = jnp.exp(scores - m)
                w = w / jnp.sum(w, axis=1, keepdims=True)
                ctx_h = lax.dot_general(
                    w, v_ref[b, :, h * DH:(h + 1) * DH],
                    (((1,), (0,)), ((), ())),
                    preferred_element_type=jnp.float32,
                )
                comm_ref[0, b * SQ:(b + 1) * SQ, h * DH:(h + 1) * DH] = (
                    ctx_h.astype(jnp.bfloat16))

        rdmas = []
        for r in (1, 2, 3):
            rdma = pltpu.make_async_remote_copy(
                src_ref=comm_ref.at[0],
                dst_ref=comm_ref.at[r],
                send_sem=send_sems.at[r - 1],
                recv_sem=recv_sems.at[r - 1],
                device_id=(lax.rem(my + r, N_DEV),),
                device_id_type=pl.DeviceIdType.MESH,
            )
            rdma.start()
            rdmas.append(rdma)

        def project(slot, origin):
            wo_slice = wo_ref[pl.ds(origin * CHUNK, CHUNK), :]
            for b in range(B):
                part = lax.dot_general(
                    comm_ref[slot, b * SQ:(b + 1) * SQ, :], wo_slice,
                    (((1,), (0,)), ((), ())),
                    preferred_element_type=jnp.float32,
                )
                if slot == 0:
                    out_ref[b] = part
                else:
                    out_ref[b] = out_ref[b] + part

        project(0, my)

        for r in (1, 3, 2):
            rdmas[r - 1].wait_recv()
            project(r, lax.rem(my + N_DEV - r, N_DEV))

        for rdma in rdmas:
            rdma.wait_send()

    return pl.pallas_call(
        body,
        out_shape=jax.ShapeDtypeStruct((B, SQ, D_MODEL), jnp.float32),
        in_specs=[pl.BlockSpec(memory_space=pltpu.VMEM)] * 5,
        out_specs=pl.BlockSpec(memory_space=pltpu.VMEM),
        scratch_shapes=[
            pltpu.VMEM((N_DEV, B * SQ, CHUNK), jnp.bfloat16),
            pltpu.SemaphoreType.DMA((N_DEV - 1,)),
            pltpu.SemaphoreType.DMA((N_DEV - 1,)),
        ],
        compiler_params=pltpu.CompilerParams(collective_id=0),
    )(x, Wq, K2, V2, Wo16)


# device time: 37317 ns/iter; 2.6941x vs baseline; 1.3438x over previous
import jax
import jax.numpy as jnp
from jax import lax
from jax.experimental import pallas as pl
from jax.experimental.pallas import tpu as pltpu

N_DEV = 4
B = 2
SQ = 512
SKV = 512
HQ_PER = 8
DH = 64
D_MODEL = 768
CHUNK = HQ_PER * DH
BLK = 64
HALF = SQ // 2
ROWS = B * SQ
QROWS = ROWS // N_DEV
HQROWS = QROWS // 2
NEG = -1e9
BF = jnp.bfloat16


def kernel(x, Wq, K_ext, V_ext, Wo):
    K2 = K_ext.reshape(B, SKV, CHUNK).astype(BF)
    V2 = V_ext.reshape(B, SKV, CHUNK)
    x16 = x.astype(BF)
    Wq16 = Wq.astype(BF)
    Wo16 = Wo.astype(BF)

    def body(x_ref, wq_ref, k_ref, v_ref, wo_ref, out_ref,
             ctx_scr, own_scr, q_buf, rs_buf, ag_buf,
             rs_send, rs_recv, ag_send, ag_recv):
        my = lax.axis_index("i")

        barrier_sem = pltpu.get_barrier_semaphore()
        for r in (1, 2, 3):
            pl.semaphore_signal(
                barrier_sem, inc=1,
                device_id=(lax.rem(my + r, N_DEV),),
                device_id_type=pl.DeviceIdType.MESH,
            )
        pl.semaphore_wait(barrier_sem, N_DEV - 1)

        row_blk = lax.broadcasted_iota(jnp.int32, (SQ, SKV), 0) // BLK
        col_blk = lax.broadcasted_iota(jnp.int32, (SQ, SKV), 1) // BLK
        mask = col_blk <= row_blk

        wo_mine = wo_ref[pl.ds(my * CHUNK, CHUNK), :]

        def ctx_part(q_part, k_part, v_part, m_part):
            s = lax.dot_general(
                q_part, k_part, (((1,), (1,)), ((), ())),
                preferred_element_type=jnp.float32,
            )
            w = jnp.exp(jnp.where(m_part, s, NEG))
            inv = 1.0 / jnp.sum(w, axis=1, keepdims=True)
            return lax.dot_general(
                w, v_part, (((1,), (0,)), ((), ())),
                preferred_element_type=jnp.float32,
            ) * inv

        def partial_rows(row0):
            return lax.dot_general(
                ctx_scr[pl.ds(row0, QROWS), :], wo_mine,
                (((1,), (0,)), ((), ())),
                preferred_element_type=jnp.float32,
            )

        def rs_quarter(q):
            part = partial_rows(q * QROWS)
            s = lax.rem(q - my + N_DEV, N_DEV)

            @pl.when(s == 0)
            def _():
                own_scr[...] = part

            @pl.when(s != 0)
            def _():
                q_buf[0, q] = part[:HQROWS].astype(BF)
                q_buf[1, q] = part[HQROWS:].astype(BF)
                for hh in range(2):
                    pltpu.make_async_remote_copy(
                        src_ref=q_buf.at[hh, q],
                        dst_ref=rs_buf.at[hh, s - 1],
                        send_sem=rs_send.at[hh, q],
                        recv_sem=rs_recv.at[hh, s - 1],
                        device_id=(q,),
                        device_id_type=pl.DeviceIdType.MESH,
                    ).start()

        for b in range(B):
            q_all = (lax.dot_general(
                x_ref[b], wq_ref[:, pl.ds(my * CHUNK, CHUNK)],
                (((1,), (0,)), ((), ())),
                preferred_element_type=jnp.float32,
            ) * 0.125).astype(BF)
            for h in range(HQ_PER):
                hs = slice(h * DH, (h + 1) * DH)
                ctx_t = ctx_part(q_all[:HALF, hs], k_ref[b, :HALF, hs],
                                 v_ref[b, :HALF, hs], mask[:HALF, :HALF])
                ctx_scr[b * SQ:b * SQ + HALF, hs] = ctx_t.astype(BF)
            rs_quarter(2 * b)
            for h in range(HQ_PER):
                hs = slice(h * DH, (h + 1) * DH)
                ctx_b = ctx_part(q_all[HALF:, hs], k_ref[b, :, hs],
                                 v_ref[b, :, hs], mask[HALF:, :])
                ctx_scr[b * SQ + HALF:(b + 1) * SQ, hs] = ctx_b.astype(BF)
            rs_quarter(2 * b + 1)

        def rs_recv_desc(hh, i):
            return pltpu.make_async_remote_copy(
                src_ref=q_buf.at[hh, 0], dst_ref=rs_buf.at[hh, i],
                send_sem=rs_send.at[hh, 0], recv_sem=rs_recv.at[hh, i],
                device_id=(0,), device_id_type=pl.DeviceIdType.MESH,
            )

        ag_rdmas = []
        for hh in range(2):
            h0 = hh * HQROWS
            for i in (0, 2, 1):
                rs_recv_desc(hh, i).wait_recv()
            tot = (own_scr[h0:h0 + HQROWS, :]
                   + rs_buf[hh, 0].astype(jnp.float32)
                   + rs_buf[hh, 1].astype(jnp.float32)
                   + rs_buf[hh, 2].astype(jnp.float32))
            ag_buf[hh, my] = tot.astype(BF)
            for r in (1, 2, 3):
                rdma = pltpu.make_async_remote_copy(
                    src_ref=ag_buf.at[hh, my],
                    dst_ref=ag_buf.at[hh, my],
                    send_sem=ag_send.at[hh, r - 1],
                    recv_sem=ag_recv.at[hh, r - 1],
                    device_id=(lax.rem(my + r, N_DEV),),
                    device_id_type=pl.DeviceIdType.MESH,
                )
                rdma.start()
                ag_rdmas.append(rdma)

        for rdma in ag_rdmas:
            rdma.wait_recv()

        for q in range(N_DEV):
            b, s0 = divmod(q * QROWS, SQ)
            out_ref[b, s0:s0 + HQROWS, :] = ag_buf[0, q]
            out_ref[b, s0 + HQROWS:s0 + QROWS, :] = ag_buf[1, q]

        for q in range(N_DEV):
            s = lax.rem(q - my + N_DEV, N_DEV)

            @pl.when(s != 0)
            def _():
                for hh in range(2):
                    pltpu.make_async_remote_copy(
                        src_ref=q_buf.at[hh, q], dst_ref=rs_buf.at[hh, 0],
                        send_sem=rs_send.at[hh, q],
                        recv_sem=rs_recv.at[hh, 0],
                        device_id=(0,), device_id_type=pl.DeviceIdType.MESH,
                    ).wait_send()

        for rdma in ag_rdmas:
            rdma.wait_send()

    return pl.pallas_call(
        body,
        out_shape=jax.ShapeDtypeStruct((B, SQ, D_MODEL), BF),
        in_specs=[pl.BlockSpec(memory_space=pltpu.VMEM)] * 5,
        out_specs=pl.BlockSpec(memory_space=pltpu.VMEM),
        scratch_shapes=[
            pltpu.VMEM((ROWS, CHUNK), BF),
            pltpu.VMEM((QROWS, D_MODEL), jnp.float32),
            pltpu.VMEM((2, N_DEV, HQROWS, D_MODEL), BF),
            pltpu.VMEM((2, N_DEV - 1, HQROWS, D_MODEL), BF),
            pltpu.VMEM((2, N_DEV, HQROWS, D_MODEL), BF),
            pltpu.SemaphoreType.DMA((2, N_DEV)),
            pltpu.SemaphoreType.DMA((2, N_DEV - 1)),
            pltpu.SemaphoreType.DMA((2, N_DEV - 1)),
            pltpu.SemaphoreType.DMA((2, N_DEV - 1)),
        ],
        compiler_params=pltpu.CompilerParams(collective_id=0),
    )(x16, Wq16, K2, V2, Wo16)
